# baseline (speedup 1.0000x reference)
import functools

import jax
import jax.numpy as jnp
from jax import lax
from jax.experimental import pallas as pl
from jax.experimental.pallas import tpu as pltpu
from jax.experimental.pallas import tpu_sc as plsc


def kernel(input, indices):
    B, C, H, W = input.shape
    num_cores = 2
    per_core = C // num_cores
    mesh = plsc.ScalarSubcoreMesh(axis_name="core", num_cores=num_cores)

    @functools.partial(
        pl.kernel,
        out_type=jax.ShapeDtypeStruct(input.shape, input.dtype),
        mesh=mesh,
        scratch_types=[
            pltpu.SMEM((C,), jnp.int32),
            pltpu.SemaphoreType.DMA,
        ],
    )
    def run(in_hbm, idx_hbm, out_hbm, idx_smem, sem):
        pltpu.sync_copy(idx_hbm, idx_smem)
        base = lax.axis_index("core") * per_core

        def issue(i, carry):
            c = base + i
            src = idx_smem[c]
            pltpu.make_async_copy(
                in_hbm.at[:, src], out_hbm.at[:, c], sem
            ).start()
            return carry

        lax.fori_loop(0, per_core, issue, 0)

        def drain(i, carry):
            pltpu.make_async_copy(
                in_hbm.at[:, 0], out_hbm.at[:, 0], sem
            ).wait()
            return carry

        lax.fori_loop(0, per_core, drain, 0)

    return run(input, indices)
